# baseline (speedup 1.0000x reference)
import jax
import jax.numpy as jnp
from jax import lax
from jax.experimental import pallas as pl
from jax.experimental.pallas import tpu as pltpu

N = 32768
B = 16
IN_DIM = 64
HIDDEN = 512
OUT4 = 256 * 4

TILE = 4096
NTILES = N // TILE

_NEG = float("-inf")


def _body(offs_s, x_ref, w1_ref, b1_hbm, w2_hbm, b2_hbm,
          out_ref, pooled_ref, w2_v, b1_v, b2_v, sem_w2, sem_b1, sem_b2):
    i = pl.program_id(0)

    @pl.when(i == 0)
    def _init():
        pooled_ref[...] = jnp.full((B, HIDDEN), _NEG, dtype=jnp.float32)
        pltpu.make_async_copy(w2_hbm, w2_v, sem_w2).start()
        pltpu.make_async_copy(b1_hbm, b1_v, sem_b1).start()
        pltpu.make_async_copy(b2_hbm, b2_v, sem_b2).start()

    h = jnp.dot(x_ref[...].astype(jnp.bfloat16), w1_ref[...].astype(jnp.bfloat16),
                preferred_element_type=jnp.float32)

    tstart = i * TILE
    lo = jnp.int32(0)
    hi = jnp.int32(0)
    for s in range(B):
        lo = lo + (offs_s[s + 1] <= tstart).astype(jnp.int32)
        hi = hi + (offs_s[s + 1] <= tstart + (TILE - 1)).astype(jnp.int32)

    rowid = lax.broadcasted_iota(jnp.int32, (TILE, 1), 0) + tstart
    prow = lax.broadcasted_iota(jnp.int32, (B, 1), 0)

    @pl.when(lo == hi)
    def _single_segment():
        seg = jnp.max(h, axis=0, keepdims=True)
        upd = jnp.where(prow == lo, jnp.broadcast_to(seg, (B, HIDDEN)), _NEG)
        pooled_ref[...] = jnp.maximum(pooled_ref[...], upd)

    @pl.when(lo != hi)
    def _multi_segment():
        def seg_step(s, carry):
            m = (rowid >= offs_s[s]) & (rowid < offs_s[s + 1])
            seg = jnp.max(jnp.where(m, h, _NEG), axis=0, keepdims=True)
            upd = jnp.where(prow == s, jnp.broadcast_to(seg, (B, HIDDEN)), _NEG)
            pooled_ref[...] = jnp.maximum(pooled_ref[...], upd)
            return carry

        lax.fori_loop(lo, hi + 1, seg_step, 0)

    @pl.when(i == NTILES - 1)
    def _finish():
        pltpu.make_async_copy(w2_hbm, w2_v, sem_w2).wait()
        pltpu.make_async_copy(b1_hbm, b1_v, sem_b1).wait()
        pltpu.make_async_copy(b2_hbm, b2_v, sem_b2).wait()
        pooled = pooled_ref[...] + b1_v[...]
        out_ref[...] = jnp.dot(pooled, w2_v[...],
                               preferred_element_type=jnp.float32) + b2_v[...]


@jax.jit
def _encode(flat_pts, batch_idx, W1, b1, W2, b2):
    idx = batch_idx.astype(jnp.int32)
    offs = jnp.sum(jnp.arange(B + 1, dtype=jnp.int32)[:, None] > idx[None, :],
                   axis=1, dtype=jnp.int32)

    grid_spec = pltpu.PrefetchScalarGridSpec(
        num_scalar_prefetch=1,
        grid=(NTILES,),
        in_specs=[
            pl.BlockSpec((TILE, IN_DIM), lambda i, *_: (i, 0)),
            pl.BlockSpec((IN_DIM, HIDDEN), lambda i, *_: (0, 0)),
            pl.BlockSpec(memory_space=pl.ANY),
            pl.BlockSpec(memory_space=pl.ANY),
            pl.BlockSpec(memory_space=pl.ANY),
        ],
        out_specs=pl.BlockSpec((B, OUT4), lambda i, *_: (0, 0)),
        scratch_shapes=[
            pltpu.VMEM((B, HIDDEN), jnp.float32),
            pltpu.VMEM((HIDDEN, OUT4), jnp.float32),
            pltpu.VMEM((1, HIDDEN), jnp.float32),
            pltpu.VMEM((1, OUT4), jnp.float32),
            pltpu.SemaphoreType.DMA,
            pltpu.SemaphoreType.DMA,
            pltpu.SemaphoreType.DMA,
        ],
    )

    proj = pl.pallas_call(
        _body,
        grid_spec=grid_spec,
        out_shape=jax.ShapeDtypeStruct((B, OUT4), jnp.float32),
        compiler_params=pltpu.CompilerParams(
            dimension_semantics=("arbitrary",),
        ),
    )(offs, flat_pts, W1, b1.reshape(1, HIDDEN), W2, b2.reshape(1, OUT4))
    return proj.reshape(B, OUT4 // 4, 4)


def kernel(flat_pts, batch_idx, W1, b1, W2, b2):
    return _encode(flat_pts, batch_idx, W1, b1, W2, b2)


# speedup vs baseline: 1.2065x; 1.2065x over previous
import jax
import jax.numpy as jnp
from jax import lax
from jax.experimental import pallas as pl
from jax.experimental.pallas import tpu as pltpu

N = 32768
B = 16
IN_DIM = 64
HIDDEN = 512
OUT4 = 256 * 4

TILE = 2048
NTILES = N // TILE

_NEG = float("-inf")


def _body(offs_s, x_ref, w1_ref, b1_hbm, w2_hbm, b2_hbm,
          out_ref, pooled_ref, ha_ref, hb_ref, w2_v, b1_v, b2_v,
          sem_w2, sem_b1, sem_b2):
    i = pl.program_id(0)

    @pl.when(i == 0)
    def _init():
        pooled_ref[...] = jnp.full((B, HIDDEN), _NEG, dtype=jnp.float32)
        pltpu.make_async_copy(w2_hbm, w2_v, sem_w2).start()
        pltpu.make_async_copy(b1_hbm, b1_v, sem_b1).start()
        pltpu.make_async_copy(b2_hbm, b2_v, sem_b2).start()

    prow = lax.broadcasted_iota(jnp.int32, (B, 1), 0)

    def _dot():
        return jnp.dot(x_ref[...].astype(jnp.bfloat16),
                       w1_ref[...].astype(jnp.bfloat16),
                       preferred_element_type=jnp.float32)

    def _reduce(j, h_ref):
        tstart = j * TILE
        lo = jnp.int32(0)
        hi = jnp.int32(0)
        for s in range(B):
            lo = lo + (offs_s[s + 1] <= tstart).astype(jnp.int32)
            hi = hi + (offs_s[s + 1] <= tstart + (TILE - 1)).astype(jnp.int32)

        h = h_ref[...]
        rowid = lax.broadcasted_iota(jnp.int32, (TILE, 1), 0) + tstart

        mA = rowid < offs_s[lo + 1]
        segA = jnp.max(jnp.where(mA, h, _NEG), axis=0, keepdims=True)
        updA = jnp.where(prow == lo, jnp.broadcast_to(segA, (B, HIDDEN)), _NEG)
        pooled_ref[...] = jnp.maximum(pooled_ref[...], updA)

        @pl.when(hi > lo)
        def _pass_b():
            mB = (rowid >= offs_s[lo + 1]) & (rowid < offs_s[lo + 2])
            segB = jnp.max(jnp.where(mB, h, _NEG), axis=0, keepdims=True)
            updB = jnp.where(prow == lo + 1,
                             jnp.broadcast_to(segB, (B, HIDDEN)), _NEG)
            pooled_ref[...] = jnp.maximum(pooled_ref[...], updB)

        @pl.when(hi > lo + 1)
        def _pass_rest():
            def seg_step(s, carry):
                m = (rowid >= offs_s[s]) & (rowid < offs_s[s + 1])
                seg = jnp.max(jnp.where(m, h, _NEG), axis=0, keepdims=True)
                upd = jnp.where(prow == s,
                                jnp.broadcast_to(seg, (B, HIDDEN)), _NEG)
                pooled_ref[...] = jnp.maximum(pooled_ref[...], upd)
                return carry

            lax.fori_loop(lo + 2, hi + 1, seg_step, 0)

    @pl.when(i % 2 == 0)
    def _even():
        ha_ref[...] = _dot()

        @pl.when(i > 0)
        def _r():
            _reduce(i - 1, hb_ref)

    @pl.when(i % 2 == 1)
    def _odd():
        hb_ref[...] = _dot()
        _reduce(i - 1, ha_ref)

    @pl.when(i == NTILES - 1)
    def _finish():
        _reduce(NTILES - 1, hb_ref if (NTILES - 1) % 2 else ha_ref)
        pltpu.make_async_copy(w2_hbm, w2_v, sem_w2).wait()
        pltpu.make_async_copy(b1_hbm, b1_v, sem_b1).wait()
        pltpu.make_async_copy(b2_hbm, b2_v, sem_b2).wait()
        pooled = pooled_ref[...] + b1_v[...]
        out_ref[...] = jnp.dot(pooled, w2_v[...],
                               preferred_element_type=jnp.float32) + b2_v[...]


@jax.jit
def _encode(flat_pts, batch_idx, W1, b1, W2, b2):
    idx = batch_idx.astype(jnp.int32)
    offs = jnp.sum(jnp.arange(B + 1, dtype=jnp.int32)[:, None] > idx[None, :],
                   axis=1, dtype=jnp.int32)

    grid_spec = pltpu.PrefetchScalarGridSpec(
        num_scalar_prefetch=1,
        grid=(NTILES,),
        in_specs=[
            pl.BlockSpec((TILE, IN_DIM), lambda i, *_: (i, 0)),
            pl.BlockSpec((IN_DIM, HIDDEN), lambda i, *_: (0, 0)),
            pl.BlockSpec(memory_space=pl.ANY),
            pl.BlockSpec(memory_space=pl.ANY),
            pl.BlockSpec(memory_space=pl.ANY),
        ],
        out_specs=pl.BlockSpec((B, OUT4), lambda i, *_: (0, 0)),
        scratch_shapes=[
            pltpu.VMEM((B, HIDDEN), jnp.float32),
            pltpu.VMEM((TILE, HIDDEN), jnp.float32),
            pltpu.VMEM((TILE, HIDDEN), jnp.float32),
            pltpu.VMEM((HIDDEN, OUT4), jnp.float32),
            pltpu.VMEM((1, HIDDEN), jnp.float32),
            pltpu.VMEM((1, OUT4), jnp.float32),
            pltpu.SemaphoreType.DMA,
            pltpu.SemaphoreType.DMA,
            pltpu.SemaphoreType.DMA,
        ],
    )

    proj = pl.pallas_call(
        _body,
        grid_spec=grid_spec,
        out_shape=jax.ShapeDtypeStruct((B, OUT4), jnp.float32),
        compiler_params=pltpu.CompilerParams(
            dimension_semantics=("arbitrary",),
        ),
    )(offs, flat_pts, W1, b1.reshape(1, HIDDEN), W2, b2.reshape(1, OUT4))
    return proj.reshape(B, OUT4 // 4, 4)


def kernel(flat_pts, batch_idx, W1, b1, W2, b2):
    return _encode(flat_pts, batch_idx, W1, b1, W2, b2)
